# Initial kernel scaffold; baseline (speedup 1.0000x reference)
import jax
import jax.numpy as jnp
from jax.experimental import pallas as pl


def kernel(feat, clusters, proj_W, proj_g, proj_b, lw_W, lw_g, lw_b, wt_W, adp_W, fuse_W, fuse_g, fuse_b, conv_W1, conv_W2, cbn1_g, cbn1_b, cbn2_g, cbn2_b, pairs_in, pairs_out, koff):
    raise NotImplementedError("write your pallas kernel here")


# speedup vs baseline: 10.6284x; 10.6284x over previous
import jax
import jax.numpy as jnp
from jax.experimental import pallas as pl
from jax.experimental.pallas import tpu as pltpu

F32 = jnp.float32
K = 4096
NKER = 27
EPS_BN = 1e-5


def _lr(x):
    return jnp.where(x >= 0, x, 0.01 * x)


def _onehot(cl, k):
    return (cl[:, None] == jax.lax.broadcasted_iota(jnp.int32, (cl.shape[0], k), 1)).astype(F32)


def _dotT(a, b):
    return jax.lax.dot_general(a, b, (((1,), (1,)), ((), ())), preferred_element_type=F32)


def _dotC(a, b):
    return jax.lax.dot_general(a, b, (((0,), (0,)), ((), ())), preferred_element_type=F32)


def _dot(a, b):
    return jax.lax.dot_general(a, b, (((1,), (0,)), ((), ())), preferred_element_type=F32)


def _gram_k(x_ref, s_ref, g_ref):
    i = pl.program_id(0)

    @pl.when(i == 0)
    def _():
        s_ref[...] = jnp.zeros_like(s_ref)
        g_ref[...] = jnp.zeros_like(g_ref)

    x = x_ref[...]
    s_ref[...] += jnp.sum(x, 0, keepdims=True)
    g_ref[...] += _dotC(x, x)


def _stats_k(x_ref, s_ref, q_ref):
    i = pl.program_id(0)

    @pl.when(i == 0)
    def _():
        s_ref[...] = jnp.zeros_like(s_ref)
        q_ref[...] = jnp.zeros_like(q_ref)

    x = x_ref[...]
    s_ref[...] += jnp.sum(x, 0, keepdims=True)
    q_ref[...] += jnp.sum(x * x, 0, keepdims=True)


def _pw_k(x_ref, cl_ref, w_ref, sc_ref, sh_ref, pw_ref, seg_ref, cnt_ref):
    i = pl.program_id(0)

    @pl.when(i == 0)
    def _():
        seg_ref[...] = jnp.zeros_like(seg_ref)
        cnt_ref[...] = jnp.zeros_like(cnt_ref)

    x = x_ref[...]
    pw = _lr(_dotT(x, w_ref[...]) * sc_ref[...] + sh_ref[...])
    pw_ref[...] = pw
    oh = _onehot(cl_ref[0, 0, :], K)
    seg_ref[...] += _dotC(oh, pw)
    cnt_ref[...] += _dotC(oh, jnp.ones((x.shape[0], 1), F32))


def _mproj_k(seg_ref, cnt_ref, w_ref, out_ref):
    m = seg_ref[...] / jnp.maximum(cnt_ref[...], 1.0)
    out_ref[...] = _dotT(m, w_ref[...])


def _t_k(pw_ref, cl_ref, mp_ref, w_ref, t_ref, mx_ref):
    i = pl.program_id(0)

    @pl.when(i == 0)
    def _():
        mx_ref[...] = jnp.full_like(mx_ref, -1e30)

    oh = _onehot(cl_ref[0, 0, :], K)
    t = _dotT(pw_ref[...], w_ref[...]) - _dot(oh, mp_ref[...])
    t_ref[...] = t
    mx_ref[...] = jnp.maximum(mx_ref[...], jnp.max(t))


def _esum_k(t_ref, cl_ref, mx_ref, seg2_ref):
    i = pl.program_id(0)

    @pl.when(i == 0)
    def _():
        seg2_ref[...] = jnp.zeros_like(seg2_ref)

    e = jnp.exp(t_ref[...] - mx_ref[...])
    oh = _onehot(cl_ref[0, 0, :], K)
    seg2_ref[...] += _dotC(oh, e)


def _pf_k(t_ref, cl_ref, mx_ref, seg2_ref, x_ref, wp_ref, sc_ref, sh_ref, segpf_ref):
    i = pl.program_id(0)

    @pl.when(i == 0)
    def _():
        segpf_ref[...] = jnp.zeros_like(segpf_ref)

    oh = _onehot(cl_ref[0, 0, :], K)
    e = jnp.exp(t_ref[...] - mx_ref[...])
    den = _dot(oh, seg2_ref[...]) + 1e-6
    pw3 = e / den
    pf = _lr(_dotT(x_ref[...], wp_ref[...]) * sc_ref[...] + sh_ref[...]) * pw3
    segpf_ref[...] += _dotC(oh, pf)


def _comb_k(x_ref, cl_ref, s0_ref, s1_ref, s2_ref, adp_ref, ab_ref,
            w3_ref, sc3_ref, sh3_ref, fa_ref, fb_ref,
            y_ref, ys_ref, yq_ref):
    i = pl.program_id(0)

    @pl.when(i == 0)
    def _():
        ys_ref[...] = jnp.zeros_like(ys_ref)
        yq_ref[...] = jnp.zeros_like(yq_ref)

    x = x_ref[...]
    lg = _dotT(x, adp_ref[...]) + ab_ref[...]
    m = jnp.max(lg, 1, keepdims=True)
    e = jnp.exp(lg - m)
    a = e / jnp.sum(e, 1, keepdims=True)
    fagg = jnp.zeros((x.shape[0], x.shape[1]), F32)
    for j, s_ref in enumerate((s0_ref, s1_ref, s2_ref)):
        oh = _onehot(cl_ref[j, 0, 0, :], K)
        fagg += a[:, j:j + 1] * _dot(oh, s_ref[...])
    f2 = _lr(_dotT(x, w3_ref[...]) * sc3_ref[...] + sh3_ref[...])
    y = _dotT(f2, fa_ref[...]) + _dotT(fagg, fb_ref[...])
    y_ref[...] = y
    ys_ref[...] += jnp.sum(y, 0, keepdims=True)
    yq_ref[...] += jnp.sum(y * y, 0, keepdims=True)


def _fuse_apply_k(y_ref, x_ref, sc_ref, sh_ref, o_ref):
    o_ref[...] = _lr(y_ref[...] * sc_ref[...] + sh_ref[...]) + x_ref[...]


def _aff_lrelu_k(x_ref, sc_ref, sh_ref, o_ref):
    o_ref[...] = _lr(x_ref[...] * sc_ref[...] + sh_ref[...])


def _final_k(h_ref, r_ref, sc_ref, sh_ref, o_ref):
    o_ref[...] = _lr(h_ref[...] * sc_ref[...] + sh_ref[...] + r_ref[...])


def _conv_k(koff_ref, xg_ref, w_ref, o_ref):
    b = pl.program_id(0)
    bp = xg_ref.shape[0]
    g0 = b * bp
    g1 = g0 + bp - 1
    k0 = jnp.int32(0)
    k1 = jnp.int32(0)
    for k in range(1, NKER):
        k0 += (koff_ref[k] <= g0).astype(jnp.int32)
        k1 += (koff_ref[k] <= g1).astype(jnp.int32)
    xg = xg_ref[...]
    y0 = _dotT(xg, w_ref[k0])
    y1 = _dotT(xg, w_ref[k1])
    bnd = koff_ref[k0 + 1]
    gvec = g0 + jax.lax.broadcasted_iota(jnp.int32, (bp, 1), 0)
    o_ref[...] = jnp.where(gvec < bnd, y0, y1)


def _row_specs(nb, b, c):
    return pl.BlockSpec((b, c), lambda i: (i, 0))


def kernel(feat, clusters, proj_W, proj_g, proj_b, lw_W, lw_g, lw_b, wt_W, adp_W, fuse_W, fuse_g, fuse_b, conv_W1, conv_W2, cbn1_g, cbn1_b, cbn2_g, cbn2_b, pairs_in, pairs_out, koff):
    n, c = feat.shape
    B = 400
    nb = n // B
    row = _row_specs(nb, B, c)
    full_cc = pl.BlockSpec((c, c), lambda i: (0, 0))
    full_1c = pl.BlockSpec((1, c), lambda i: (0, 0))
    full_kc = pl.BlockSpec((K, c), lambda i: (0, 0))
    full_k1 = pl.BlockSpec((K, 1), lambda i: (0, 0))
    one1 = pl.BlockSpec((1, 1), lambda i: (0, 0))
    cl_spec = pl.BlockSpec((1, 1, B), lambda i: (i, 0, 0))
    sd = jax.ShapeDtypeStruct

    s, g = pl.pallas_call(
        _gram_k, grid=(nb,),
        in_specs=[row],
        out_specs=[full_1c, full_cc],
        out_shape=[sd((1, c), F32), sd((c, c), F32)],
    )(feat)
    mu = s / n
    e2 = g / n

    def bn_affine(w, gg, bb):
        m = (mu @ w.T)[0]
        v = jnp.einsum('oc,cd,od->o', w, e2, w) - m * m
        sc = gg / jnp.sqrt(v + EPS_BN)
        sh = bb - m * sc
        return sc.reshape(1, c), sh.reshape(1, c)

    feats_seg = []
    cl_r = clusters.reshape(3, nb, 1, B)
    for i in range(3):
        sc_lw, sh_lw = bn_affine(lw_W[i], lw_g[i], lw_b[i])
        pw, seg, cnt = pl.pallas_call(
            _pw_k, grid=(nb,),
            in_specs=[row, cl_spec, full_cc, full_1c, full_1c],
            out_specs=[row, full_kc, full_k1],
            out_shape=[sd((n, c), F32), sd((K, c), F32), sd((K, 1), F32)],
        )(feat, cl_r[i], lw_W[i], sc_lw, sh_lw)

        mproj = pl.pallas_call(
            _mproj_k, grid=(1,),
            in_specs=[full_kc, full_k1, full_cc],
            out_specs=full_kc,
            out_shape=sd((K, c), F32),
        )(seg, cnt, wt_W[i])

        t, mx = pl.pallas_call(
            _t_k, grid=(nb,),
            in_specs=[row, cl_spec, full_kc, full_cc],
            out_specs=[row, one1],
            out_shape=[sd((n, c), F32), sd((1, 1), F32)],
        )(pw, cl_r[i], mproj, wt_W[i])

        seg2 = pl.pallas_call(
            _esum_k, grid=(nb,),
            in_specs=[row, cl_spec, one1],
            out_specs=full_kc,
            out_shape=sd((K, c), F32),
        )(t, cl_r[i], mx)

        sc_p, sh_p = bn_affine(proj_W[i], proj_g[i], proj_b[i])
        segpf = pl.pallas_call(
            _pf_k, grid=(nb,),
            in_specs=[row, cl_spec, one1, full_kc, row, full_cc, full_1c, full_1c],
            out_specs=full_kc,
            out_shape=sd((K, c), F32),
        )(t, cl_r[i], mx, seg2, feat, proj_W[i], sc_p, sh_p)
        feats_seg.append(segpf)

    BC = 200
    nbc = n // BC
    rowc = pl.BlockSpec((BC, c), lambda i: (i, 0))
    cl_c = pl.BlockSpec((3, 1, 1, BC), lambda i: (0, i, 0, 0))
    adp_p = jnp.pad(adp_W, ((0, 5), (0, 0)))
    abias = jnp.concatenate([jnp.zeros((1, 3), F32), jnp.full((1, 5), -1e30, F32)], 1)
    full_8c = pl.BlockSpec((8, c), lambda i: (0, 0))
    full_18 = pl.BlockSpec((1, 8), lambda i: (0, 0))
    kc = pl.BlockSpec((K, c), lambda i: (0, 0))
    sc3, sh3 = bn_affine(proj_W[3], proj_g[3], proj_b[3])
    y, ys, yq = pl.pallas_call(
        _comb_k, grid=(nbc,),
        in_specs=[rowc, cl_c, kc, kc, kc, full_8c, full_18,
                  pl.BlockSpec((c, c), lambda i: (0, 0)), pl.BlockSpec((1, c), lambda i: (0, 0)),
                  pl.BlockSpec((1, c), lambda i: (0, 0)),
                  pl.BlockSpec((c, c), lambda i: (0, 0)), pl.BlockSpec((c, c), lambda i: (0, 0))],
        out_specs=[rowc, pl.BlockSpec((1, c), lambda i: (0, 0)), pl.BlockSpec((1, c), lambda i: (0, 0))],
        out_shape=[sd((n, c), F32), sd((1, c), F32), sd((1, c), F32)],
    )(feat, clusters.reshape(3, nbc, 1, BC), feats_seg[0], feats_seg[1], feats_seg[2],
      adp_p, abias, proj_W[3], sc3, sh3, fuse_W[:, :c], fuse_W[:, c:])

    def stats_affine(s_, q_, gg, bb):
        m = s_[0] / n
        v = q_[0] / n - m * m
        sc = gg / jnp.sqrt(v + EPS_BN)
        sh = bb - m * sc
        return sc.reshape(1, c), sh.reshape(1, c)

    sc_f, sh_f = stats_affine(ys, yq, fuse_g, fuse_b)
    fused = pl.pallas_call(
        _fuse_apply_k, grid=(nb,),
        in_specs=[row, row, full_1c, full_1c],
        out_specs=row,
        out_shape=sd((n, c), F32),
    )(y, feat, sc_f, sh_f)

    p = pairs_in.shape[0]
    BP = 512
    npb = -(-p // BP)
    pad = npb * BP - p
    pin_p = jnp.pad(pairs_in, (0, pad))
    pout_p = jnp.pad(pairs_out, (0, pad), constant_values=n)
    koff32 = jnp.asarray(koff).astype(jnp.int32)
    rowp = pl.BlockSpec((BP, c), lambda i: (i, 0))
    wspec = pl.BlockSpec((NKER, c, c), lambda i: (0, 0, 0))
    smem = pl.BlockSpec(memory_space=pltpu.SMEM)

    def subm(x, w):
        xg = x[pin_p]
        v = pl.pallas_call(
            _conv_k, grid=(npb,),
            in_specs=[smem, rowp, wspec],
            out_specs=rowp,
            out_shape=sd((npb * BP, c), F32),
        )(koff32, xg, w)
        return jnp.zeros((n, c), F32).at[pout_p].add(v, mode='drop')

    def bn_stats(x):
        s_, q_ = pl.pallas_call(
            _stats_k, grid=(nb,),
            in_specs=[row],
            out_specs=[full_1c, full_1c],
            out_shape=[sd((1, c), F32), sd((1, c), F32)],
        )(x)
        return s_, q_

    h = subm(fused, conv_W1)
    s1, q1 = bn_stats(h)
    sc1, sh1 = stats_affine(s1, q1, cbn1_g, cbn1_b)
    h1n = pl.pallas_call(
        _aff_lrelu_k, grid=(nb,),
        in_specs=[row, full_1c, full_1c],
        out_specs=row,
        out_shape=sd((n, c), F32),
    )(h, sc1, sh1)

    h2 = subm(h1n, conv_W2)
    s2, q2 = bn_stats(h2)
    sc2, sh2 = stats_affine(s2, q2, cbn2_g, cbn2_b)
    out = pl.pallas_call(
        _final_k, grid=(nb,),
        in_specs=[row, row, full_1c, full_1c],
        out_specs=row,
        out_shape=sd((n, c), F32),
    )(h2, fused, sc2, sh2)
    return out


# speedup vs baseline: 10.7754x; 1.0138x over previous
import jax
import jax.numpy as jnp
from jax.experimental import pallas as pl
from jax.experimental.pallas import tpu as pltpu

F32 = jnp.float32
K = 4096
NKER = 27
EPS_BN = 1e-5


def _lr(x):
    return jnp.where(x >= 0, x, 0.01 * x)


def _onehot(cl, k):
    return (cl[:, None] == jax.lax.broadcasted_iota(jnp.int32, (cl.shape[0], k), 1)).astype(jnp.bfloat16)


def _dotCb(a, b):
    return jax.lax.dot_general(a, b.astype(jnp.bfloat16), (((0,), (0,)), ((), ())),
                               preferred_element_type=F32)


def _dotb(a, b):
    return jax.lax.dot_general(a, b.astype(jnp.bfloat16), (((1,), (0,)), ((), ())),
                               preferred_element_type=F32)


def _dotT(a, b):
    return jax.lax.dot_general(a, b, (((1,), (1,)), ((), ())), preferred_element_type=F32)


def _dotC(a, b):
    return jax.lax.dot_general(a, b, (((0,), (0,)), ((), ())), preferred_element_type=F32)


def _dot(a, b):
    return jax.lax.dot_general(a, b, (((1,), (0,)), ((), ())), preferred_element_type=F32)


def _gram_k(x_ref, s_ref, g_ref):
    i = pl.program_id(0)

    @pl.when(i == 0)
    def _():
        s_ref[...] = jnp.zeros_like(s_ref)
        g_ref[...] = jnp.zeros_like(g_ref)

    x = x_ref[...]
    s_ref[...] += jnp.sum(x, 0, keepdims=True)
    g_ref[...] += _dotC(x, x)


def _stats_k(x_ref, s_ref, q_ref):
    i = pl.program_id(0)

    @pl.when(i == 0)
    def _():
        s_ref[...] = jnp.zeros_like(s_ref)
        q_ref[...] = jnp.zeros_like(q_ref)

    x = x_ref[...]
    s_ref[...] += jnp.sum(x, 0, keepdims=True)
    q_ref[...] += jnp.sum(x * x, 0, keepdims=True)


def _pw_k(x_ref, cl_ref, w_ref, sc_ref, sh_ref, pw_ref, seg_ref, cnt_ref):
    i = pl.program_id(0)

    @pl.when(i == 0)
    def _():
        seg_ref[...] = jnp.zeros_like(seg_ref)
        cnt_ref[...] = jnp.zeros_like(cnt_ref)

    x = x_ref[...]
    pw = _lr(_dotT(x, w_ref[...]) * sc_ref[...] + sh_ref[...])
    pw_ref[...] = pw
    oh = _onehot(cl_ref[0, 0, :], K)
    seg_ref[...] += _dotCb(oh, pw)
    cnt_ref[...] += _dotCb(oh, jnp.ones((x.shape[0], 1), F32))


def _mproj_k(seg_ref, cnt_ref, w_ref, out_ref):
    m = seg_ref[...] / jnp.maximum(cnt_ref[...], 1.0)
    out_ref[...] = _dotT(m, w_ref[...])


def _t_k(pw_ref, cl_ref, mp_ref, w_ref, t_ref, mx_ref):
    i = pl.program_id(0)

    @pl.when(i == 0)
    def _():
        mx_ref[...] = jnp.full_like(mx_ref, -1e30)

    oh = _onehot(cl_ref[0, 0, :], K)
    t = _dotT(pw_ref[...], w_ref[...]) - _dotb(oh, mp_ref[...])
    t_ref[...] = t
    mx_ref[...] = jnp.maximum(mx_ref[...], jnp.max(t))


def _esum_k(t_ref, cl_ref, mx_ref, seg2_ref):
    i = pl.program_id(0)

    @pl.when(i == 0)
    def _():
        seg2_ref[...] = jnp.zeros_like(seg2_ref)

    e = jnp.exp(t_ref[...] - mx_ref[...])
    oh = _onehot(cl_ref[0, 0, :], K)
    seg2_ref[...] += _dotCb(oh, e)


def _pf_k(t_ref, cl_ref, mx_ref, seg2_ref, x_ref, wp_ref, sc_ref, sh_ref, segpf_ref):
    i = pl.program_id(0)

    @pl.when(i == 0)
    def _():
        segpf_ref[...] = jnp.zeros_like(segpf_ref)

    oh = _onehot(cl_ref[0, 0, :], K)
    e = jnp.exp(t_ref[...] - mx_ref[...])
    den = _dotb(oh, seg2_ref[...]) + 1e-6
    pw3 = e / den
    pf = _lr(_dotT(x_ref[...], wp_ref[...]) * sc_ref[...] + sh_ref[...]) * pw3
    segpf_ref[...] += _dotCb(oh, pf)


def _comb_k(x_ref, cl_ref, s0_ref, s1_ref, s2_ref, adp_ref, ab_ref,
            w3_ref, sc3_ref, sh3_ref, fa_ref, fb_ref,
            y_ref, ys_ref, yq_ref):
    i = pl.program_id(0)

    @pl.when(i == 0)
    def _():
        ys_ref[...] = jnp.zeros_like(ys_ref)
        yq_ref[...] = jnp.zeros_like(yq_ref)

    x = x_ref[...]
    lg = _dotT(x, adp_ref[...]) + ab_ref[...]
    m = jnp.max(lg, 1, keepdims=True)
    e = jnp.exp(lg - m)
    a = e / jnp.sum(e, 1, keepdims=True)
    fagg = jnp.zeros((x.shape[0], x.shape[1]), F32)
    for j, s_ref in enumerate((s0_ref, s1_ref, s2_ref)):
        oh = _onehot(cl_ref[j, 0, 0, :], K)
        fagg += a[:, j:j + 1] * _dotb(oh, s_ref[...])
    f2 = _lr(_dotT(x, w3_ref[...]) * sc3_ref[...] + sh3_ref[...])
    y = _dotT(f2, fa_ref[...]) + _dotT(fagg, fb_ref[...])
    y_ref[...] = y
    ys_ref[...] += jnp.sum(y, 0, keepdims=True)
    yq_ref[...] += jnp.sum(y * y, 0, keepdims=True)


def _fuse_apply_k(y_ref, x_ref, sc_ref, sh_ref, o_ref):
    o_ref[...] = _lr(y_ref[...] * sc_ref[...] + sh_ref[...]) + x_ref[...]


def _aff_lrelu_k(x_ref, sc_ref, sh_ref, o_ref):
    o_ref[...] = _lr(x_ref[...] * sc_ref[...] + sh_ref[...])


def _final_k(h_ref, r_ref, sc_ref, sh_ref, o_ref):
    o_ref[...] = _lr(h_ref[...] * sc_ref[...] + sh_ref[...] + r_ref[...])


def _conv_k(koff_ref, xg_ref, w_ref, o_ref):
    b = pl.program_id(0)
    bp = xg_ref.shape[0]
    g0 = b * bp
    g1 = g0 + bp - 1
    k0 = jnp.int32(0)
    k1 = jnp.int32(0)
    for k in range(1, NKER):
        k0 += (koff_ref[k] <= g0).astype(jnp.int32)
        k1 += (koff_ref[k] <= g1).astype(jnp.int32)
    xg = xg_ref[...]
    y0 = _dotT(xg, w_ref[k0])
    y1 = _dotT(xg, w_ref[k1])
    bnd = koff_ref[k0 + 1]
    gvec = g0 + jax.lax.broadcasted_iota(jnp.int32, (bp, 1), 0)
    o_ref[...] = jnp.where(gvec < bnd, y0, y1)


def _row_specs(nb, b, c):
    return pl.BlockSpec((b, c), lambda i: (i, 0))


def kernel(feat, clusters, proj_W, proj_g, proj_b, lw_W, lw_g, lw_b, wt_W, adp_W, fuse_W, fuse_g, fuse_b, conv_W1, conv_W2, cbn1_g, cbn1_b, cbn2_g, cbn2_b, pairs_in, pairs_out, koff):
    n, c = feat.shape
    B = 400
    nb = n // B
    row = _row_specs(nb, B, c)
    full_cc = pl.BlockSpec((c, c), lambda i: (0, 0))
    full_1c = pl.BlockSpec((1, c), lambda i: (0, 0))
    full_kc = pl.BlockSpec((K, c), lambda i: (0, 0))
    full_k1 = pl.BlockSpec((K, 1), lambda i: (0, 0))
    one1 = pl.BlockSpec((1, 1), lambda i: (0, 0))
    cl_spec = pl.BlockSpec((1, 1, B), lambda i: (i, 0, 0))
    sd = jax.ShapeDtypeStruct

    s, g = pl.pallas_call(
        _gram_k, grid=(nb,),
        in_specs=[row],
        out_specs=[full_1c, full_cc],
        out_shape=[sd((1, c), F32), sd((c, c), F32)],
    )(feat)
    mu = s / n
    e2 = g / n

    def bn_affine(w, gg, bb):
        m = (mu @ w.T)[0]
        v = jnp.einsum('oc,cd,od->o', w, e2, w) - m * m
        sc = gg / jnp.sqrt(v + EPS_BN)
        sh = bb - m * sc
        return sc.reshape(1, c), sh.reshape(1, c)

    feats_seg = []
    cl_r = clusters.reshape(3, nb, 1, B)
    for i in range(3):
        sc_lw, sh_lw = bn_affine(lw_W[i], lw_g[i], lw_b[i])
        pw, seg, cnt = pl.pallas_call(
            _pw_k, grid=(nb,),
            in_specs=[row, cl_spec, full_cc, full_1c, full_1c],
            out_specs=[row, full_kc, full_k1],
            out_shape=[sd((n, c), F32), sd((K, c), F32), sd((K, 1), F32)],
        )(feat, cl_r[i], lw_W[i], sc_lw, sh_lw)

        mproj = pl.pallas_call(
            _mproj_k, grid=(1,),
            in_specs=[full_kc, full_k1, full_cc],
            out_specs=full_kc,
            out_shape=sd((K, c), F32),
        )(seg, cnt, wt_W[i])

        t, mx = pl.pallas_call(
            _t_k, grid=(nb,),
            in_specs=[row, cl_spec, full_kc, full_cc],
            out_specs=[row, one1],
            out_shape=[sd((n, c), F32), sd((1, 1), F32)],
        )(pw, cl_r[i], mproj, wt_W[i])

        seg2 = pl.pallas_call(
            _esum_k, grid=(nb,),
            in_specs=[row, cl_spec, one1],
            out_specs=full_kc,
            out_shape=sd((K, c), F32),
        )(t, cl_r[i], mx)

        sc_p, sh_p = bn_affine(proj_W[i], proj_g[i], proj_b[i])
        segpf = pl.pallas_call(
            _pf_k, grid=(nb,),
            in_specs=[row, cl_spec, one1, full_kc, row, full_cc, full_1c, full_1c],
            out_specs=full_kc,
            out_shape=sd((K, c), F32),
        )(t, cl_r[i], mx, seg2, feat, proj_W[i], sc_p, sh_p)
        feats_seg.append(segpf)

    BC = 200
    nbc = n // BC
    rowc = pl.BlockSpec((BC, c), lambda i: (i, 0))
    cl_c = pl.BlockSpec((3, 1, 1, BC), lambda i: (0, i, 0, 0))
    adp_p = jnp.pad(adp_W, ((0, 5), (0, 0)))
    abias = jnp.concatenate([jnp.zeros((1, 3), F32), jnp.full((1, 5), -1e30, F32)], 1)
    full_8c = pl.BlockSpec((8, c), lambda i: (0, 0))
    full_18 = pl.BlockSpec((1, 8), lambda i: (0, 0))
    kc = pl.BlockSpec((K, c), lambda i: (0, 0))
    sc3, sh3 = bn_affine(proj_W[3], proj_g[3], proj_b[3])
    y, ys, yq = pl.pallas_call(
        _comb_k, grid=(nbc,),
        in_specs=[rowc, cl_c, kc, kc, kc, full_8c, full_18,
                  pl.BlockSpec((c, c), lambda i: (0, 0)), pl.BlockSpec((1, c), lambda i: (0, 0)),
                  pl.BlockSpec((1, c), lambda i: (0, 0)),
                  pl.BlockSpec((c, c), lambda i: (0, 0)), pl.BlockSpec((c, c), lambda i: (0, 0))],
        out_specs=[rowc, pl.BlockSpec((1, c), lambda i: (0, 0)), pl.BlockSpec((1, c), lambda i: (0, 0))],
        out_shape=[sd((n, c), F32), sd((1, c), F32), sd((1, c), F32)],
    )(feat, clusters.reshape(3, nbc, 1, BC), feats_seg[0], feats_seg[1], feats_seg[2],
      adp_p, abias, proj_W[3], sc3, sh3, fuse_W[:, :c], fuse_W[:, c:])

    def stats_affine(s_, q_, gg, bb):
        m = s_[0] / n
        v = q_[0] / n - m * m
        sc = gg / jnp.sqrt(v + EPS_BN)
        sh = bb - m * sc
        return sc.reshape(1, c), sh.reshape(1, c)

    sc_f, sh_f = stats_affine(ys, yq, fuse_g, fuse_b)
    fused = pl.pallas_call(
        _fuse_apply_k, grid=(nb,),
        in_specs=[row, row, full_1c, full_1c],
        out_specs=row,
        out_shape=sd((n, c), F32),
    )(y, feat, sc_f, sh_f)

    p = pairs_in.shape[0]
    BP = 512
    npb = -(-p // BP)
    pad = npb * BP - p
    pin_p = jnp.pad(pairs_in, (0, pad))
    pout_p = jnp.pad(pairs_out, (0, pad), constant_values=n)
    koff32 = jnp.asarray(koff).astype(jnp.int32)
    rowp = pl.BlockSpec((BP, c), lambda i: (i, 0))
    wspec = pl.BlockSpec((NKER, c, c), lambda i: (0, 0, 0))
    smem = pl.BlockSpec(memory_space=pltpu.SMEM)

    def subm(x, w):
        xg = x[pin_p]
        v = pl.pallas_call(
            _conv_k, grid=(npb,),
            in_specs=[smem, rowp, wspec],
            out_specs=rowp,
            out_shape=sd((npb * BP, c), F32),
        )(koff32, xg, w)
        return jnp.zeros((n, c), F32).at[pout_p].add(v, mode='drop')

    def bn_stats(x):
        s_, q_ = pl.pallas_call(
            _stats_k, grid=(nb,),
            in_specs=[row],
            out_specs=[full_1c, full_1c],
            out_shape=[sd((1, c), F32), sd((1, c), F32)],
        )(x)
        return s_, q_

    h = subm(fused, conv_W1)
    s1, q1 = bn_stats(h)
    sc1, sh1 = stats_affine(s1, q1, cbn1_g, cbn1_b)
    h1n = pl.pallas_call(
        _aff_lrelu_k, grid=(nb,),
        in_specs=[row, full_1c, full_1c],
        out_specs=row,
        out_shape=sd((n, c), F32),
    )(h, sc1, sh1)

    h2 = subm(h1n, conv_W2)
    s2, q2 = bn_stats(h2)
    sc2, sh2 = stats_affine(s2, q2, cbn2_g, cbn2_b)
    out = pl.pallas_call(
        _final_k, grid=(nb,),
        in_specs=[row, row, full_1c, full_1c],
        out_specs=row,
        out_shape=sd((n, c), F32),
    )(h2, fused, sc2, sh2)
    return out


# speedup vs baseline: 10.9466x; 1.0159x over previous
import jax
import jax.numpy as jnp
from jax.experimental import pallas as pl
from jax.experimental.pallas import tpu as pltpu

F32 = jnp.float32
K = 4096
NKER = 27
EPS_BN = 1e-5


def _lr(x):
    return jnp.where(x >= 0, x, 0.01 * x)


def _onehot(cl, k):
    return (cl[:, None] == jax.lax.broadcasted_iota(jnp.int32, (cl.shape[0], k), 1)).astype(jnp.bfloat16)


def _dotCb(a, b):
    return jax.lax.dot_general(a, b.astype(jnp.bfloat16), (((0,), (0,)), ((), ())),
                               preferred_element_type=F32)


def _dotb(a, b):
    return jax.lax.dot_general(a, b.astype(jnp.bfloat16), (((1,), (0,)), ((), ())),
                               preferred_element_type=F32)


def _dotT(a, b):
    return jax.lax.dot_general(a, b, (((1,), (1,)), ((), ())), preferred_element_type=F32)


def _dotC(a, b):
    return jax.lax.dot_general(a, b, (((0,), (0,)), ((), ())), preferred_element_type=F32)


def _dot(a, b):
    return jax.lax.dot_general(a, b, (((1,), (0,)), ((), ())), preferred_element_type=F32)


def _gram_k(x_ref, s_ref, g_ref):
    i = pl.program_id(0)

    @pl.when(i == 0)
    def _():
        s_ref[...] = jnp.zeros_like(s_ref)
        g_ref[...] = jnp.zeros_like(g_ref)

    x = x_ref[...]
    s_ref[...] += jnp.sum(x, 0, keepdims=True)
    g_ref[...] += _dotC(x, x)


def _stats_k(x_ref, s_ref, q_ref):
    i = pl.program_id(0)

    @pl.when(i == 0)
    def _():
        s_ref[...] = jnp.zeros_like(s_ref)
        q_ref[...] = jnp.zeros_like(q_ref)

    x = x_ref[...].astype(F32)
    s_ref[...] += jnp.sum(x, 0, keepdims=True)
    q_ref[...] += jnp.sum(x * x, 0, keepdims=True)


def _pw_k(x_ref, cl_ref, w_ref, sc_ref, sh_ref, pw_ref, seg_ref, cnt_ref):
    i = pl.program_id(0)

    @pl.when(i == 0)
    def _():
        seg_ref[...] = jnp.zeros_like(seg_ref)
        cnt_ref[...] = jnp.zeros_like(cnt_ref)

    x = x_ref[...]
    pw = _lr(_dotT(x, w_ref[...]) * sc_ref[...] + sh_ref[...])
    pw_ref[...] = pw
    oh = _onehot(cl_ref[0, 0, :], K)
    seg_ref[...] += _dotCb(oh, pw)
    cnt_ref[...] += _dotCb(oh, jnp.ones((x.shape[0], 1), F32))


def _mproj_k(seg_ref, cnt_ref, w_ref, out_ref):
    m = seg_ref[...] / jnp.maximum(cnt_ref[...], 1.0)
    out_ref[...] = _dotT(m, w_ref[...])


def _t_k(pw_ref, cl_ref, mp_ref, w_ref, t_ref, mx_ref):
    i = pl.program_id(0)

    @pl.when(i == 0)
    def _():
        mx_ref[...] = jnp.full_like(mx_ref, -1e30)

    oh = _onehot(cl_ref[0, 0, :], K)
    t = _dotT(pw_ref[...], w_ref[...]) - _dotb(oh, mp_ref[...])
    t_ref[...] = t
    mx_ref[...] = jnp.maximum(mx_ref[...], jnp.max(t))


def _esum_k(t_ref, cl_ref, mx_ref, seg2_ref):
    i = pl.program_id(0)

    @pl.when(i == 0)
    def _():
        seg2_ref[...] = jnp.zeros_like(seg2_ref)

    e = jnp.exp(t_ref[...] - mx_ref[...])
    oh = _onehot(cl_ref[0, 0, :], K)
    seg2_ref[...] += _dotCb(oh, e)


def _pf_k(t_ref, cl_ref, mx_ref, seg2_ref, x_ref, wp_ref, sc_ref, sh_ref, segpf_ref):
    i = pl.program_id(0)

    @pl.when(i == 0)
    def _():
        segpf_ref[...] = jnp.zeros_like(segpf_ref)

    oh = _onehot(cl_ref[0, 0, :], K)
    e = jnp.exp(t_ref[...] - mx_ref[...])
    den = _dotb(oh, seg2_ref[...]) + 1e-6
    pw3 = e / den
    pf = _lr(_dotT(x_ref[...], wp_ref[...]) * sc_ref[...] + sh_ref[...]) * pw3
    segpf_ref[...] += _dotCb(oh, pf)


def _comb_k(x_ref, cl_ref, s0_ref, s1_ref, s2_ref, adp_ref, ab_ref,
            w3_ref, sc3_ref, sh3_ref, fa_ref, fb_ref,
            y_ref, ys_ref, yq_ref):
    i = pl.program_id(0)

    @pl.when(i == 0)
    def _():
        ys_ref[...] = jnp.zeros_like(ys_ref)
        yq_ref[...] = jnp.zeros_like(yq_ref)

    x = x_ref[...]
    lg = _dotT(x, adp_ref[...]) + ab_ref[...]
    m = jnp.max(lg, 1, keepdims=True)
    e = jnp.exp(lg - m)
    a = e / jnp.sum(e, 1, keepdims=True)
    fagg = jnp.zeros((x.shape[0], x.shape[1]), F32)
    for j, s_ref in enumerate((s0_ref, s1_ref, s2_ref)):
        oh = _onehot(cl_ref[j, 0, 0, :], K)
        fagg += a[:, j:j + 1] * _dotb(oh, s_ref[...])
    f2 = _lr(_dotT(x, w3_ref[...]) * sc3_ref[...] + sh3_ref[...])
    y = _dotT(f2, fa_ref[...]) + _dotT(fagg, fb_ref[...])
    y_ref[...] = y
    ys_ref[...] += jnp.sum(y, 0, keepdims=True)
    yq_ref[...] += jnp.sum(y * y, 0, keepdims=True)


def _fuse_apply_k(y_ref, x_ref, sc_ref, sh_ref, o_ref, ob_ref):
    v = _lr(y_ref[...] * sc_ref[...] + sh_ref[...]) + x_ref[...]
    o_ref[...] = v
    ob_ref[...] = v.astype(jnp.bfloat16)


def _aff_lrelu_k(x_ref, sc_ref, sh_ref, o_ref):
    r = _lr(x_ref[...].astype(F32) * sc_ref[...] + sh_ref[...])
    o_ref[...] = r.astype(o_ref.dtype)


def _final_k(h_ref, r_ref, sc_ref, sh_ref, o_ref):
    o_ref[...] = _lr(h_ref[...].astype(F32) * sc_ref[...] + sh_ref[...] + r_ref[...])


def _conv_k(koff_ref, xg_ref, w_ref, o_ref):
    b = pl.program_id(0)
    bp = xg_ref.shape[0]
    g0 = b * bp
    g1 = g0 + bp - 1
    k0 = jnp.int32(0)
    k1 = jnp.int32(0)
    for k in range(1, NKER):
        k0 += (koff_ref[k] <= g0).astype(jnp.int32)
        k1 += (koff_ref[k] <= g1).astype(jnp.int32)
    xg = xg_ref[...]
    y0 = _dotT(xg, w_ref[k0].astype(xg.dtype))
    y1 = _dotT(xg, w_ref[k1].astype(xg.dtype))
    bnd = koff_ref[k0 + 1]
    gvec = g0 + jax.lax.broadcasted_iota(jnp.int32, (bp, 1), 0)
    o_ref[...] = jnp.where(gvec < bnd, y0, y1).astype(o_ref.dtype)


def _row_specs(nb, b, c):
    return pl.BlockSpec((b, c), lambda i: (i, 0))


def kernel(feat, clusters, proj_W, proj_g, proj_b, lw_W, lw_g, lw_b, wt_W, adp_W, fuse_W, fuse_g, fuse_b, conv_W1, conv_W2, cbn1_g, cbn1_b, cbn2_g, cbn2_b, pairs_in, pairs_out, koff):
    n, c = feat.shape
    B = 400
    nb = n // B
    row = _row_specs(nb, B, c)
    full_cc = pl.BlockSpec((c, c), lambda i: (0, 0))
    full_1c = pl.BlockSpec((1, c), lambda i: (0, 0))
    full_kc = pl.BlockSpec((K, c), lambda i: (0, 0))
    full_k1 = pl.BlockSpec((K, 1), lambda i: (0, 0))
    one1 = pl.BlockSpec((1, 1), lambda i: (0, 0))
    cl_spec = pl.BlockSpec((1, 1, B), lambda i: (i, 0, 0))
    sd = jax.ShapeDtypeStruct

    s, g = pl.pallas_call(
        _gram_k, grid=(nb,),
        in_specs=[row],
        out_specs=[full_1c, full_cc],
        out_shape=[sd((1, c), F32), sd((c, c), F32)],
    )(feat)
    mu = s / n
    e2 = g / n

    def bn_affine(w, gg, bb):
        m = (mu @ w.T)[0]
        v = jnp.einsum('oc,cd,od->o', w, e2, w) - m * m
        sc = gg / jnp.sqrt(v + EPS_BN)
        sh = bb - m * sc
        return sc.reshape(1, c), sh.reshape(1, c)

    feats_seg = []
    cl_r = clusters.reshape(3, nb, 1, B)
    for i in range(3):
        sc_lw, sh_lw = bn_affine(lw_W[i], lw_g[i], lw_b[i])
        pw, seg, cnt = pl.pallas_call(
            _pw_k, grid=(nb,),
            in_specs=[row, cl_spec, full_cc, full_1c, full_1c],
            out_specs=[row, full_kc, full_k1],
            out_shape=[sd((n, c), F32), sd((K, c), F32), sd((K, 1), F32)],
        )(feat, cl_r[i], lw_W[i], sc_lw, sh_lw)

        mproj = pl.pallas_call(
            _mproj_k, grid=(1,),
            in_specs=[full_kc, full_k1, full_cc],
            out_specs=full_kc,
            out_shape=sd((K, c), F32),
        )(seg, cnt, wt_W[i])

        t, mx = pl.pallas_call(
            _t_k, grid=(nb,),
            in_specs=[row, cl_spec, full_kc, full_cc],
            out_specs=[row, one1],
            out_shape=[sd((n, c), F32), sd((1, 1), F32)],
        )(pw, cl_r[i], mproj, wt_W[i])

        seg2 = pl.pallas_call(
            _esum_k, grid=(nb,),
            in_specs=[row, cl_spec, one1],
            out_specs=full_kc,
            out_shape=sd((K, c), F32),
        )(t, cl_r[i], mx)

        sc_p, sh_p = bn_affine(proj_W[i], proj_g[i], proj_b[i])
        segpf = pl.pallas_call(
            _pf_k, grid=(nb,),
            in_specs=[row, cl_spec, one1, full_kc, row, full_cc, full_1c, full_1c],
            out_specs=full_kc,
            out_shape=sd((K, c), F32),
        )(t, cl_r[i], mx, seg2, feat, proj_W[i], sc_p, sh_p)
        feats_seg.append(segpf)

    BC = 200
    nbc = n // BC
    rowc = pl.BlockSpec((BC, c), lambda i: (i, 0))
    cl_c = pl.BlockSpec((3, 1, 1, BC), lambda i: (0, i, 0, 0))
    adp_p = jnp.pad(adp_W, ((0, 5), (0, 0)))
    abias = jnp.concatenate([jnp.zeros((1, 3), F32), jnp.full((1, 5), -1e30, F32)], 1)
    full_8c = pl.BlockSpec((8, c), lambda i: (0, 0))
    full_18 = pl.BlockSpec((1, 8), lambda i: (0, 0))
    kc = pl.BlockSpec((K, c), lambda i: (0, 0))
    sc3, sh3 = bn_affine(proj_W[3], proj_g[3], proj_b[3])
    y, ys, yq = pl.pallas_call(
        _comb_k, grid=(nbc,),
        in_specs=[rowc, cl_c, kc, kc, kc, full_8c, full_18,
                  pl.BlockSpec((c, c), lambda i: (0, 0)), pl.BlockSpec((1, c), lambda i: (0, 0)),
                  pl.BlockSpec((1, c), lambda i: (0, 0)),
                  pl.BlockSpec((c, c), lambda i: (0, 0)), pl.BlockSpec((c, c), lambda i: (0, 0))],
        out_specs=[rowc, pl.BlockSpec((1, c), lambda i: (0, 0)), pl.BlockSpec((1, c), lambda i: (0, 0))],
        out_shape=[sd((n, c), F32), sd((1, c), F32), sd((1, c), F32)],
    )(feat, clusters.reshape(3, nbc, 1, BC), feats_seg[0], feats_seg[1], feats_seg[2],
      adp_p, abias, proj_W[3], sc3, sh3, fuse_W[:, :c], fuse_W[:, c:])

    def stats_affine(s_, q_, gg, bb):
        m = s_[0] / n
        v = q_[0] / n - m * m
        sc = gg / jnp.sqrt(v + EPS_BN)
        sh = bb - m * sc
        return sc.reshape(1, c), sh.reshape(1, c)

    sc_f, sh_f = stats_affine(ys, yq, fuse_g, fuse_b)
    fused, fused_b = pl.pallas_call(
        _fuse_apply_k, grid=(nb,),
        in_specs=[row, row, full_1c, full_1c],
        out_specs=[row, row],
        out_shape=[sd((n, c), F32), sd((n, c), jnp.bfloat16)],
    )(y, feat, sc_f, sh_f)

    p = pairs_in.shape[0]
    BP = 512
    npb = -(-p // BP)
    pad = npb * BP - p
    pin_p = jnp.pad(pairs_in, (0, pad))
    pout_p = jnp.pad(pairs_out, (0, pad), constant_values=n)
    koff32 = jnp.asarray(koff).astype(jnp.int32)
    rowp = pl.BlockSpec((BP, c), lambda i: (i, 0))
    wspec = pl.BlockSpec((NKER, c, c), lambda i: (0, 0, 0))
    smem = pl.BlockSpec(memory_space=pltpu.SMEM)

    def subm(x, w):
        xg = x[pin_p]
        v = pl.pallas_call(
            _conv_k, grid=(npb,),
            in_specs=[smem, rowp, wspec],
            out_specs=rowp,
            out_shape=sd((npb * BP, c), jnp.bfloat16),
        )(koff32, xg, w)
        return jnp.zeros((n, c), jnp.bfloat16).at[pout_p].add(v, mode='drop')

    def bn_stats(x):
        s_, q_ = pl.pallas_call(
            _stats_k, grid=(nb,),
            in_specs=[row],
            out_specs=[full_1c, full_1c],
            out_shape=[sd((1, c), F32), sd((1, c), F32)],
        )(x)
        return s_, q_

    h = subm(fused_b, conv_W1)
    s1, q1 = bn_stats(h)
    sc1, sh1 = stats_affine(s1, q1, cbn1_g, cbn1_b)
    h1n = pl.pallas_call(
        _aff_lrelu_k, grid=(nb,),
        in_specs=[row, full_1c, full_1c],
        out_specs=row,
        out_shape=sd((n, c), jnp.bfloat16),
    )(h, sc1, sh1)

    h2 = subm(h1n, conv_W2)
    s2, q2 = bn_stats(h2)
    sc2, sh2 = stats_affine(s2, q2, cbn2_g, cbn2_b)
    out = pl.pallas_call(
        _final_k, grid=(nb,),
        in_specs=[row, row, full_1c, full_1c],
        out_specs=row,
        out_shape=sd((n, c), F32),
    )(h2, fused, sc2, sh2)
    return out
